# Initial kernel scaffold; baseline (speedup 1.0000x reference)
import jax
import jax.numpy as jnp
from jax.experimental import pallas as pl


def kernel(x, edge_index, edge_attr, u, batch, W1, b1, W2, b2):
    raise NotImplementedError("write your pallas kernel here")


# speedup vs baseline: 5.5737x; 5.5737x over previous
import functools

import jax
import jax.numpy as jnp
from jax import lax
from jax.experimental import pallas as pl
from jax.experimental.pallas import tpu as pltpu
from jax.experimental.pallas import tpu_sc as plsc

N = 100000
D = 128
B = 256
HIDDEN = 256

CHUNK = 128
NUM_FULL = N // CHUNK
TAIL = N - NUM_FULL * CHUNK
NC = 2
NS = 16
NW = NC * NS
MAX_ITERS = (NUM_FULL + NW - 1) // NW
CNT_W = 128
ROWS_PER_TILE = B // NS


def _seg_sum_body(x_hbm, e_hbm, b_hbm, node_out, edge_out, cnt_out,
                  xbuf, ebuf, idx, ones, txbuf, tebuf, tidx, tones,
                  zrow, cstage, accx, acce, accc):
    c = lax.axis_index("c")
    s = lax.axis_index("s")
    wid = c * NS + s

    zvec = jnp.zeros((16,), jnp.float32)
    ovec = jnp.ones((16,), jnp.float32)
    for i in range(ROWS_PER_TILE):
        for j in range(D // 16):
            zrow[i, pl.ds(j * 16, 16)] = zvec
    for i in range(CHUNK):
        for j in range(CNT_W // 16):
            ones[i, pl.ds(j * 16, 16)] = ovec
    for i in range(TAIL):
        for j in range(CNT_W // 16):
            tones[i, pl.ds(j * 16, 16)] = ovec
    for i in range(ROWS_PER_TILE):
        for j in range(CNT_W // 16):
            cstage[i, pl.ds(j * 16, 16)] = zvec

    base_r = s * ROWS_PER_TILE
    pltpu.sync_copy(zrow, accx.at[pl.ds(base_r, ROWS_PER_TILE)])
    pltpu.sync_copy(zrow, acce.at[pl.ds(base_r, ROWS_PER_TILE)])
    pltpu.sync_copy(cstage, accc.at[pl.ds(base_r, ROWS_PER_TILE)])
    plsc.subcore_barrier()

    def body(t, _):
        cid = wid + t * NW

        @pl.when(cid < NUM_FULL)
        def _():
            row0 = cid * CHUNK
            pltpu.sync_copy(b_hbm.at[pl.ds(row0, CHUNK)], idx)
            pltpu.sync_copy(x_hbm.at[pl.ds(row0, CHUNK)], xbuf)
            pltpu.sync_copy(e_hbm.at[pl.ds(row0, CHUNK)], ebuf)
            pltpu.sync_copy(xbuf, accx.at[idx], add=True)
            pltpu.sync_copy(ebuf, acce.at[idx], add=True)
            pltpu.sync_copy(ones, accc.at[idx], add=True)

        return 0

    lax.fori_loop(0, MAX_ITERS, body, 0)

    @pl.when(wid == 0)
    def _():
        row0 = NUM_FULL * CHUNK
        pltpu.sync_copy(b_hbm.at[pl.ds(row0, TAIL)], tidx)
        pltpu.sync_copy(x_hbm.at[pl.ds(row0, TAIL)], txbuf)
        pltpu.sync_copy(e_hbm.at[pl.ds(row0, TAIL)], tebuf)
        pltpu.sync_copy(txbuf, accx.at[tidx], add=True)
        pltpu.sync_copy(tebuf, acce.at[tidx], add=True)
        pltpu.sync_copy(tones, accc.at[tidx], add=True)

    plsc.subcore_barrier()

    pltpu.sync_copy(accx.at[pl.ds(base_r, ROWS_PER_TILE)], zrow)
    pltpu.sync_copy(zrow, node_out.at[c, pl.ds(base_r, ROWS_PER_TILE)])
    pltpu.sync_copy(acce.at[pl.ds(base_r, ROWS_PER_TILE)], zrow)
    pltpu.sync_copy(zrow, edge_out.at[c, pl.ds(base_r, ROWS_PER_TILE)])
    pltpu.sync_copy(accc.at[pl.ds(base_r, ROWS_PER_TILE)], cstage)
    pltpu.sync_copy(cstage, cnt_out.at[c, pl.ds(base_r, ROWS_PER_TILE)])


def _segment_sums(x, edge_attr, batch):
    mesh = plsc.VectorSubcoreMesh(core_axis_name="c", subcore_axis_name="s")
    return pl.kernel(
        _seg_sum_body,
        out_type=(
            jax.ShapeDtypeStruct((NC, B, D), jnp.float32),
            jax.ShapeDtypeStruct((NC, B, D), jnp.float32),
            jax.ShapeDtypeStruct((NC, B, CNT_W), jnp.float32),
        ),
        mesh=mesh,
        scratch_types=[
            pltpu.VMEM((CHUNK, D), jnp.float32),
            pltpu.VMEM((CHUNK, D), jnp.float32),
            pltpu.VMEM((CHUNK,), jnp.int32),
            pltpu.VMEM((CHUNK, CNT_W), jnp.float32),
            pltpu.VMEM((TAIL, D), jnp.float32),
            pltpu.VMEM((TAIL, D), jnp.float32),
            pltpu.VMEM((TAIL,), jnp.int32),
            pltpu.VMEM((TAIL, CNT_W), jnp.float32),
            pltpu.VMEM((ROWS_PER_TILE, D), jnp.float32),
            pltpu.VMEM((ROWS_PER_TILE, CNT_W), jnp.float32),
            pltpu.VMEM_SHARED((B, D), jnp.float32),
            pltpu.VMEM_SHARED((B, D), jnp.float32),
            pltpu.VMEM_SHARED((B, CNT_W), jnp.float32),
        ],
    )(x, edge_attr, batch)


def _mlp_body(node_ref, edge_ref, cnt_ref, w1_ref, b1_ref, w2_ref, b2_ref,
              out_ref):
    ns = node_ref[0] + node_ref[1]
    es = edge_ref[0] + edge_ref[1]
    cnt = cnt_ref[0, :, 0:1] + cnt_ref[1, :, 0:1]
    denom = cnt + 1e-6
    nm = ns / denom
    em = es / denom
    w1 = w1_ref[...]
    dn = (((1,), (1,)), ((), ()))
    h = lax.dot_general(nm, w1[:, :D], dn,
                        preferred_element_type=jnp.float32,
                        precision=lax.Precision.HIGHEST)
    h += lax.dot_general(em, w1[:, D:], dn,
                         preferred_element_type=jnp.float32,
                         precision=lax.Precision.HIGHEST)
    h = jnp.maximum(h + b1_ref[...], 0.0)
    out = lax.dot_general(h, w2_ref[...], dn,
                          preferred_element_type=jnp.float32,
                          precision=lax.Precision.HIGHEST)
    out_ref[...] = out + b2_ref[...]


def _pooled_mlp(node_sums, edge_sums, cnts, W1, b1, W2, b2):
    return pl.pallas_call(
        _mlp_body,
        out_shape=jax.ShapeDtypeStruct((B, D), jnp.float32),
    )(node_sums, edge_sums, cnts, W1, b1.reshape(1, HIDDEN), W2,
      b2.reshape(1, D))


@jax.jit
def kernel(x, edge_index, edge_attr, u, batch, W1, b1, W2, b2):
    del edge_index, u
    node_sums, edge_sums, cnts = _segment_sums(
        x, edge_attr, batch.astype(jnp.int32))
    return _pooled_mlp(node_sums, edge_sums, cnts, W1, b1, W2, b2)


# speedup vs baseline: 10.5883x; 1.8997x over previous
import jax
import jax.numpy as jnp
from jax import lax
from jax.experimental import pallas as pl
from jax.experimental.pallas import tpu as pltpu
from jax.experimental.pallas import tpu_sc as plsc

N = 100000
D = 128
B = 256
HIDDEN = 256

CHUNK = 128
NUM_FULL = N // CHUNK
TAIL = N - NUM_FULL * CHUNK
NC = 2
NS = 16
NW = NC * NS
MAX_ITERS = (NUM_FULL + NW - 1) // NW
ROWS_PER_TILE = B // NS
NPAD = NUM_FULL * CHUNK + CHUNK


def _seg_sum_body(x_hbm, e_hbm, b_hbm, node_out, edge_out,
                  xbuf0, xbuf1, ebuf0, ebuf1, idx0, idx1, tbuf, tidx,
                  zrow, accx, acce, sem0, sem1):
    c = lax.axis_index("c")
    s = lax.axis_index("s")
    wid = c * NS + s
    xbufs = (xbuf0, xbuf1)
    ebufs = (ebuf0, ebuf1)
    idxs = (idx0, idx1)
    sems = (sem0, sem1)

    zvec = jnp.zeros((16,), jnp.float32)
    for i in range(ROWS_PER_TILE):
        for j in range(D // 16):
            zrow[i, pl.ds(j * 16, 16)] = zvec

    base_r = s * ROWS_PER_TILE
    pltpu.sync_copy(zrow, accx.at[pl.ds(base_r, ROWS_PER_TILE)])
    pltpu.sync_copy(zrow, acce.at[pl.ds(base_r, ROWS_PER_TILE)])
    plsc.subcore_barrier()

    def issue(t, b):
        cid = wid + t * NW

        @pl.when(cid < NUM_FULL)
        def _():
            row0 = cid * CHUNK
            pltpu.async_copy(b_hbm.at[pl.ds(row0, CHUNK)], idxs[b], sems[b])
            pltpu.async_copy(x_hbm.at[pl.ds(row0, CHUNK)], xbufs[b], sems[b])
            pltpu.async_copy(e_hbm.at[pl.ds(row0, CHUNK)], ebufs[b], sems[b])

    def wait_and_scatter(t, b):
        cid = wid + t * NW

        @pl.when(cid < NUM_FULL)
        def _():
            pltpu.make_async_copy(b_hbm.at[pl.ds(0, CHUNK)], idxs[b],
                                  sems[b]).wait()
            pltpu.make_async_copy(x_hbm.at[pl.ds(0, CHUNK)], xbufs[b],
                                  sems[b]).wait()
            pltpu.make_async_copy(e_hbm.at[pl.ds(0, CHUNK)], ebufs[b],
                                  sems[b]).wait()
            pltpu.sync_copy(xbufs[b], accx.at[idxs[b]], add=True)
            pltpu.sync_copy(ebufs[b], acce.at[idxs[b]], add=True)

    issue(0, 0)

    def pair(t2, _):
        for b in range(2):
            t = t2 * 2 + b
            issue(t + 1, 1 - b)
            wait_and_scatter(t, b)
        return 0

    lax.fori_loop(0, MAX_ITERS // 2, pair, 0)
    wait_and_scatter(MAX_ITERS - 1, (MAX_ITERS - 1) % 2)

    @pl.when(wid == 0)
    def _():
        row0 = NUM_FULL * CHUNK
        pltpu.sync_copy(b_hbm.at[pl.ds(row0, TAIL)], tidx)
        pltpu.sync_copy(x_hbm.at[pl.ds(row0, TAIL)], tbuf)
        pltpu.sync_copy(tbuf, accx.at[tidx], add=True)
        pltpu.sync_copy(e_hbm.at[pl.ds(row0, TAIL)], tbuf)
        pltpu.sync_copy(tbuf, acce.at[tidx], add=True)

    plsc.subcore_barrier()

    pltpu.sync_copy(accx.at[pl.ds(base_r, ROWS_PER_TILE)], zrow)
    pltpu.sync_copy(zrow, node_out.at[c, pl.ds(base_r, ROWS_PER_TILE)])
    pltpu.sync_copy(acce.at[pl.ds(base_r, ROWS_PER_TILE)], zrow)
    pltpu.sync_copy(zrow, edge_out.at[c, pl.ds(base_r, ROWS_PER_TILE)])


def _segment_sums(x, edge_attr, batch):
    mesh = plsc.VectorSubcoreMesh(core_axis_name="c", subcore_axis_name="s")
    return pl.kernel(
        _seg_sum_body,
        out_type=(
            jax.ShapeDtypeStruct((NC, B, D), jnp.float32),
            jax.ShapeDtypeStruct((NC, B, D), jnp.float32),
        ),
        mesh=mesh,
        scratch_types=[
            pltpu.VMEM((CHUNK, D), jnp.float32),
            pltpu.VMEM((CHUNK, D), jnp.float32),
            pltpu.VMEM((CHUNK, D), jnp.float32),
            pltpu.VMEM((CHUNK, D), jnp.float32),
            pltpu.VMEM((CHUNK,), jnp.int32),
            pltpu.VMEM((CHUNK,), jnp.int32),
            pltpu.VMEM((TAIL, D), jnp.float32),
            pltpu.VMEM((TAIL,), jnp.int32),
            pltpu.VMEM((ROWS_PER_TILE, D), jnp.float32),
            pltpu.VMEM_SHARED((B, D), jnp.float32),
            pltpu.VMEM_SHARED((B, D), jnp.float32),
            pltpu.SemaphoreType.DMA,
            pltpu.SemaphoreType.DMA,
        ],
    )(x, edge_attr, batch)


def _hist_body(b_ref, cnt_ref):
    ids = lax.broadcasted_iota(jnp.int32, (B, CHUNK), 0)

    def step(t, acc):
        row = b_ref[pl.ds(t, 1), :]
        return acc + jnp.where(row == ids, 1.0, 0.0)

    cnt_ref[...] = lax.fori_loop(0, NPAD // CHUNK,
                                 step, jnp.zeros((B, CHUNK), jnp.float32))


def _histogram(batch_padded):
    return pl.pallas_call(
        _hist_body,
        out_shape=jax.ShapeDtypeStruct((B, CHUNK), jnp.float32),
    )(batch_padded)


def _mlp_body(node_ref, edge_ref, cnt_ref, w1_ref, b1_ref, w2_ref, b2_ref,
              out_ref):
    ns = node_ref[0] + node_ref[1]
    es = edge_ref[0] + edge_ref[1]
    cnt = jnp.sum(cnt_ref[...], axis=1, keepdims=True)
    denom = cnt + 1e-6
    nm = ns / denom
    em = es / denom
    w1 = w1_ref[...]
    dn = (((1,), (1,)), ((), ()))
    h = lax.dot_general(nm, w1[:, :D], dn,
                        preferred_element_type=jnp.float32,
                        precision=lax.Precision.HIGHEST)
    h += lax.dot_general(em, w1[:, D:], dn,
                         preferred_element_type=jnp.float32,
                         precision=lax.Precision.HIGHEST)
    h = jnp.maximum(h + b1_ref[...], 0.0)
    out = lax.dot_general(h, w2_ref[...], dn,
                          preferred_element_type=jnp.float32,
                          precision=lax.Precision.HIGHEST)
    out_ref[...] = out + b2_ref[...]


def _pooled_mlp(node_sums, edge_sums, cnts, W1, b1, W2, b2):
    return pl.pallas_call(
        _mlp_body,
        out_shape=jax.ShapeDtypeStruct((B, D), jnp.float32),
    )(node_sums, edge_sums, cnts, W1, b1.reshape(1, HIDDEN), W2,
      b2.reshape(1, D))


@jax.jit
def kernel(x, edge_index, edge_attr, u, batch, W1, b1, W2, b2):
    del edge_index, u
    batch = batch.astype(jnp.int32)
    bp = jnp.concatenate([batch, jnp.full((NPAD - N,), B, jnp.int32)])
    cnts = _histogram(bp.reshape(NPAD // CHUNK, CHUNK))
    node_sums, edge_sums = _segment_sums(x, edge_attr, batch)
    return _pooled_mlp(node_sums, edge_sums, cnts, W1, b1, W2, b2)
